# Initial kernel scaffold; baseline (speedup 1.0000x reference)
import jax
import jax.numpy as jnp
from jax.experimental import pallas as pl


def kernel(x, Wq, Wk):
    raise NotImplementedError("write your pallas kernel here")


# speedup vs baseline: 17.5466x; 17.5466x over previous
import jax
import jax.numpy as jnp
from jax import lax
from jax.experimental import pallas as pl
from jax.experimental.pallas import tpu as pltpu

D_K = 64
TOP_K = 8
SCALE = D_K ** (-0.5)
BN = 256


def _body(x_ref, wq_ref, wk_ref, out_ref, k_ref):
    nb = pl.program_id(1)

    @pl.when(nb == 0)
    def _compute_k():
        k_ref[...] = lax.dot_general(
            x_ref[0], wk_ref[...],
            dimension_numbers=(((1,), (1,)), ((), ())),
            preferred_element_type=jnp.float32)

    xb = x_ref[0, pl.ds(nb * BN, BN), :]
    q = lax.dot_general(xb, wq_ref[...],
                        dimension_numbers=(((1,), (1,)), ((), ())),
                        preferred_element_type=jnp.float32)
    s = lax.dot_general(q, k_ref[...],
                        dimension_numbers=(((1,), (1,)), ((), ())),
                        preferred_element_type=jnp.float32) * SCALE

    n = s.shape[1]
    col = lax.broadcasted_iota(jnp.int32, s.shape, 1)
    neg = jnp.float32(-jnp.inf)
    s_work = s
    acc = jnp.zeros(s.shape, jnp.float32)
    m0 = None
    ssum = None
    for k in range(TOP_K):
        m = jnp.max(s_work, axis=1, keepdims=True)
        if k == 0:
            m0 = m
            e = jnp.ones_like(m)
            ssum = e
        else:
            e = jnp.exp(m - m0)
            ssum = ssum + e
        idx = jnp.min(jnp.where(s_work == m, col, n), axis=1, keepdims=True)
        hit = col == idx
        acc = jnp.where(hit, jnp.broadcast_to(e, s.shape), acc)
        if k < TOP_K - 1:
            s_work = jnp.where(hit, neg, s_work)
    out_ref[0] = acc * (1.0 / ssum)


def kernel(x, Wq, Wk):
    B, N, C = x.shape
    return pl.pallas_call(
        _body,
        grid=(B, N // BN),
        in_specs=[
            pl.BlockSpec((1, N, C), lambda b, nb: (b, 0, 0)),
            pl.BlockSpec((D_K, C), lambda b, nb: (0, 0)),
            pl.BlockSpec((D_K, C), lambda b, nb: (0, 0)),
        ],
        out_specs=pl.BlockSpec((1, BN, N), lambda b, nb: (b, nb, 0)),
        out_shape=jax.ShapeDtypeStruct((B, N, N), jnp.float32),
        scratch_shapes=[pltpu.VMEM((N, D_K), jnp.float32)],
    )(x, Wq, Wk)


# speedup vs baseline: 41.6265x; 2.3723x over previous
import jax
import jax.numpy as jnp
from jax import lax
from jax.experimental import pallas as pl
from jax.experimental.pallas import tpu as pltpu

D_K = 64
TOP_K = 8
SCALE = D_K ** (-0.5)
BN = 256

_DN = (((1,), (1,)), ((), ()))


def _body(x_ref, wq_ref, wk_ref, out_ref, k_ref):
    nb = pl.program_id(1)

    @pl.when(nb == 0)
    def _compute_k():
        k_ref[...] = lax.dot_general(
            x_ref[0], wk_ref[...], dimension_numbers=_DN,
            preferred_element_type=jnp.float32)

    xb = x_ref[0, pl.ds(nb * BN, BN), :]
    q = lax.dot_general(xb, wq_ref[...], dimension_numbers=_DN,
                        preferred_element_type=jnp.float32) * jnp.float32(SCALE)
    s = lax.dot_general(q, k_ref[...], dimension_numbers=_DN,
                        preferred_element_type=jnp.float32)

    neg = jnp.float32(-jnp.inf)
    s_work = s
    m0 = None
    ssum = None
    t = None
    for k in range(TOP_K):
        m = jnp.max(s_work, axis=1, keepdims=True)
        if k == 0:
            m0 = m
            ssum = jnp.ones_like(m)
        else:
            ssum = ssum + jnp.exp(m - m0)
        t = m
        if k < TOP_K - 1:
            s_work = jnp.where(s_work == m, neg, s_work)
    rz = 1.0 / ssum
    out_ref[0] = jnp.where(s >= t, jnp.exp(s - m0) * rz, 0.0)


def kernel(x, Wq, Wk):
    B, N, C = x.shape
    return pl.pallas_call(
        _body,
        grid=(B, N // BN),
        in_specs=[
            pl.BlockSpec((1, N, C), lambda b, nb: (b, 0, 0)),
            pl.BlockSpec((D_K, C), lambda b, nb: (0, 0)),
            pl.BlockSpec((D_K, C), lambda b, nb: (0, 0)),
        ],
        out_specs=pl.BlockSpec((1, BN, N), lambda b, nb: (b, nb, 0)),
        out_shape=jax.ShapeDtypeStruct((B, N, N), jnp.float32),
        scratch_shapes=[pltpu.VMEM((N, D_K), jnp.float32)],
    )(x, Wq, Wk)
